# baseline (device time: 38797 ns/iter reference)
import jax
import jax.numpy as jnp
from jax import lax
from jax.experimental import pallas as pl
from jax.experimental.pallas import tpu as pltpu

B, Sq, D = 2, 128, 512
Hq, Dh = 8, 64
SCALE = 0.125
N_STAGES = 2


def kernel(x, Wq, Wo, K_ext, V_ext):
    _, Skv_loc, _, _ = K_ext.shape

    def body(x_ref, wq_ref, wo_ref, k_ref, v_ref, out_ref,
             o_buf, m_buf, l_buf, att_ref, send_sems, recv_sems):
        my = lax.axis_index("i")
        partners = [3 - my, my ^ 1]

        barrier_sem = pltpu.get_barrier_semaphore()
        for p in partners:
            pl.semaphore_signal(barrier_sem, inc=1, device_id=(p,),
                                device_id_type=pl.DeviceIdType.MESH)
        pl.semaphore_wait(barrier_sem, 2)

        for b in range(B):
            q_b = jnp.dot(x_ref[b], wq_ref[...],
                          preferred_element_type=jnp.float32)
            for h in range(Hq):
                q_bh = q_b[:, h * Dh:(h + 1) * Dh]
                k_bh = k_ref[b, :, h, :]
                v_bh = v_ref[b, :, h, :]
                s = lax.dot_general(
                    q_bh, k_bh, (((1,), (1,)), ((), ())),
                    preferred_element_type=jnp.float32) * SCALE
                m_loc = jnp.max(s, axis=-1)
                p_att = jnp.exp(s - m_loc[:, None])
                l_loc = jnp.sum(p_att, axis=-1)
                o_buf[0, b, h] = jnp.dot(p_att, v_bh,
                                         preferred_element_type=jnp.float32)
                m_buf[0, b, h] = m_loc
                l_buf[0, b, h] = l_loc

        for stage in range(N_STAGES):
            slot = stage + 1
            copies = []
            for t, buf in enumerate((o_buf, m_buf, l_buf)):
                c = pltpu.make_async_remote_copy(
                    src_ref=buf.at[0],
                    dst_ref=buf.at[slot],
                    send_sem=send_sems.at[stage, t],
                    recv_sem=recv_sems.at[stage, t],
                    device_id=(partners[stage],),
                    device_id_type=pl.DeviceIdType.MESH,
                )
                c.start()
                copies.append(c)
            for c in copies:
                c.wait()
            for b in range(B):
                for h in range(Hq):
                    m0 = m_buf[0, b, h]
                    m1 = m_buf[slot, b, h]
                    mn = jnp.maximum(m0, m1)
                    a0 = jnp.exp(m0 - mn)
                    a1 = jnp.exp(m1 - mn)
                    l_buf[0, b, h] = l_buf[0, b, h] * a0 + l_buf[slot, b, h] * a1
                    o_buf[0, b, h] = (o_buf[0, b, h] * a0[:, None]
                                      + o_buf[slot, b, h] * a1[:, None])
                    m_buf[0, b, h] = mn

        for b in range(B):
            for h in range(Hq):
                att_ref[b, :, h * Dh:(h + 1) * Dh] = (
                    o_buf[0, b, h] / l_buf[0, b, h][:, None])
            out_ref[b] = jnp.dot(att_ref[b], wo_ref[...],
                                 preferred_element_type=jnp.float32)

    n_slots = N_STAGES + 1
    return pl.pallas_call(
        body,
        out_shape=jax.ShapeDtypeStruct((B, Sq, D), jnp.float32),
        in_specs=[pl.BlockSpec(memory_space=pltpu.VMEM)] * 5,
        out_specs=pl.BlockSpec(memory_space=pltpu.VMEM),
        scratch_shapes=[
            pltpu.VMEM((n_slots, B, Hq, Sq, Dh), jnp.float32),
            pltpu.VMEM((n_slots, B, Hq, Sq), jnp.float32),
            pltpu.VMEM((n_slots, B, Hq, Sq), jnp.float32),
            pltpu.VMEM((B, Sq, D), jnp.float32),
            pltpu.SemaphoreType.DMA((N_STAGES, 3)),
            pltpu.SemaphoreType.DMA((N_STAGES, 3)),
        ],
        compiler_params=pltpu.CompilerParams(collective_id=0),
    )(x, Wq, Wo, K_ext, V_ext)
